# baseline (speedup 1.0000x reference)
import functools

import jax
import jax.numpy as jnp
from jax.experimental import pallas as pl

_NUM_VECTOR = 8
_LAYER_IDX = 16
_BLK = 512


def _copy_route_kernel(x_ref, w_ref, b_ref, vp_ref, out_ref, *, nblk, blk):
    j = pl.program_id(1)
    out_ref[...] = x_ref[...]

    @pl.when(j == nblk - 1)
    def _route():
        token_act = x_ref[0, blk - 1, :].reshape(1, -1)
        scores = jnp.dot(token_act, w_ref[...],
                         preferred_element_type=jnp.float32) + b_ref[...]
        probs = jax.nn.softmax(scores, axis=-1)
        idx = jnp.argmax(probs[0, :])
        keep = idx == _NUM_VECTOR
        onehot = (jax.lax.broadcasted_iota(jnp.int32, (1, _NUM_VECTOR), 1)
                  == jnp.minimum(idx, _NUM_VECTOR - 1)).astype(jnp.float32)
        replacement = jnp.dot(onehot, vp_ref[...],
                              preferred_element_type=jnp.float32)
        out_ref[0, blk - 1, :] = jnp.where(keep, token_act, replacement)[0]


def kernel(x, vector_pool, gate_W, gate_b):
    B, S, H = x.shape
    vp16 = vector_pool[:, _LAYER_IDX, :]
    gate_b2 = gate_b.reshape(1, -1)
    nblk = S // _BLK
    grid = (B, nblk)
    return pl.pallas_call(
        functools.partial(_copy_route_kernel, nblk=nblk, blk=_BLK),
        grid=grid,
        in_specs=[
            pl.BlockSpec((1, _BLK, H), lambda b, j: (b, j, 0)),
            pl.BlockSpec((H, _NUM_VECTOR + 1), lambda b, j: (0, 0)),
            pl.BlockSpec((1, _NUM_VECTOR + 1), lambda b, j: (0, 0)),
            pl.BlockSpec((_NUM_VECTOR, H), lambda b, j: (0, 0)),
        ],
        out_specs=pl.BlockSpec((1, _BLK, H), lambda b, j: (b, j, 0)),
        out_shape=jax.ShapeDtypeStruct((B, S, H), x.dtype),
    )(x, gate_W, gate_b2, vp16)


# speedup vs baseline: 1.0217x; 1.0217x over previous
import functools

import jax
import jax.numpy as jnp
from jax.experimental import pallas as pl

_NUM_VECTOR = 8
_LAYER_IDX = 16
_BLK = 1024


def _copy_route_kernel(x_ref, w_ref, b_ref, vp_ref, out_ref, *, nblk, blk):
    j = pl.program_id(1)
    out_ref[...] = x_ref[...]

    @pl.when(j == nblk - 1)
    def _route():
        token_act = x_ref[0, blk - 1, :].reshape(1, -1)
        scores = jnp.dot(token_act, w_ref[...],
                         preferred_element_type=jnp.float32) + b_ref[...]
        probs = jax.nn.softmax(scores, axis=-1)
        idx = jnp.argmax(probs[0, :])
        keep = idx == _NUM_VECTOR
        onehot = (jax.lax.broadcasted_iota(jnp.int32, (1, _NUM_VECTOR), 1)
                  == jnp.minimum(idx, _NUM_VECTOR - 1)).astype(jnp.float32)
        replacement = jnp.dot(onehot, vp_ref[...],
                              preferred_element_type=jnp.float32)
        out_ref[0, blk - 1, :] = jnp.where(keep, token_act, replacement)[0]


def kernel(x, vector_pool, gate_W, gate_b):
    B, S, H = x.shape
    vp16 = vector_pool[:, _LAYER_IDX, :]
    gate_b2 = gate_b.reshape(1, -1)
    nblk = S // _BLK
    grid = (B, nblk)
    return pl.pallas_call(
        functools.partial(_copy_route_kernel, nblk=nblk, blk=_BLK),
        grid=grid,
        in_specs=[
            pl.BlockSpec((1, _BLK, H), lambda b, j: (b, j, 0)),
            pl.BlockSpec((H, _NUM_VECTOR + 1), lambda b, j: (0, 0)),
            pl.BlockSpec((1, _NUM_VECTOR + 1), lambda b, j: (0, 0)),
            pl.BlockSpec((_NUM_VECTOR, H), lambda b, j: (0, 0)),
        ],
        out_specs=pl.BlockSpec((1, _BLK, H), lambda b, j: (b, j, 0)),
        out_shape=jax.ShapeDtypeStruct((B, S, H), x.dtype),
    )(x, gate_W, gate_b2, vp16)
